# Initial kernel scaffold; baseline (speedup 1.0000x reference)
import functools

import jax
import jax.numpy as jnp
from jax import lax
from jax.experimental import pallas as pl
from jax.experimental.pallas import tpu as pltpu
from jax.experimental.pallas import tpu_sc as plsc

N = 10000
F = 128
E = 320000
NC = 2
NS = 16
NW = NC * NS
EPT = E // NW
BLK = 80
NB = EPT // BLK
RPS = N // NS
DW = 16

_mesh = plsc.VectorSubcoreMesh(core_axis_name="c", subcore_axis_name="s")


def _deg_body(dst_hbm, out_hbm, dst_v, ones_v, zeros_v, accd):
    c = lax.axis_index("c")
    s = lax.axis_index("s")
    wid = s * NC + c

    def fill(i, _):
        ones_v[i, :] = jnp.ones((16,), jnp.float32)
        zeros_v[i, :] = jnp.zeros((16,), jnp.float32)
        return 0

    lax.fori_loop(0, BLK, fill, 0)

    base = s * RPS
    for k in range(7):
        pltpu.sync_copy(zeros_v, accd.at[pl.ds(base + k * BLK, BLK)])
    pltpu.sync_copy(zeros_v.at[pl.ds(0, RPS - 7 * BLK)],
                    accd.at[pl.ds(base + 7 * BLK, RPS - 7 * BLK)])
    plsc.subcore_barrier()

    pltpu.sync_copy(dst_hbm.at[wid], dst_v)

    def body(j, _):
        pltpu.sync_copy(ones_v, accd.at[dst_v.at[j]], add=True)
        return 0

    lax.fori_loop(0, NB, body, 0)
    plsc.subcore_barrier()
    pltpu.sync_copy(accd.at[pl.ds(base, RPS)], out_hbm.at[c, pl.ds(base, RPS)])


_deg_call = functools.partial(
    pl.kernel,
    out_type=jax.ShapeDtypeStruct((NC, N, DW), jnp.float32),
    mesh=_mesh,
    scratch_types=[
        pltpu.VMEM((NB, BLK), jnp.int32),
        pltpu.VMEM((BLK, DW), jnp.float32),
        pltpu.VMEM((BLK, DW), jnp.float32),
        pltpu.VMEM_SHARED((N, DW), jnp.float32),
    ],
)(_deg_body)


def _edge_body(src_hbm, dst_hbm, g_hbm, out_hbm, src_v, dst_v, rows, zrows, acc,
               sem):
    c = lax.axis_index("c")
    s = lax.axis_index("s")
    wid = s * NC + c

    def zfill(r, _):
        for k in range(F // 16):
            zrows[r, pl.ds(k * 16, 16)] = jnp.zeros((16,), jnp.float32)
        return 0

    lax.fori_loop(0, BLK, zfill, 0)

    base = s * RPS
    for k in range(7):
        pltpu.sync_copy(zrows, acc.at[pl.ds(base + k * BLK, BLK)])
    pltpu.sync_copy(zrows.at[pl.ds(0, RPS - 7 * BLK)],
                    acc.at[pl.ds(base + 7 * BLK, RPS - 7 * BLK)])

    pltpu.sync_copy(src_hbm.at[wid], src_v)
    pltpu.sync_copy(dst_hbm.at[wid], dst_v)
    plsc.subcore_barrier()

    def body(j, _):
        pltpu.async_copy(g_hbm.at[src_v.at[j]], rows, sem).wait()
        pltpu.sync_copy(rows, acc.at[dst_v.at[j]], add=True)
        return 0

    lax.fori_loop(0, NB, body, 0)
    plsc.subcore_barrier()
    pltpu.sync_copy(acc.at[pl.ds(base, RPS)], out_hbm.at[c, pl.ds(base, RPS)])


_edge_call = functools.partial(
    pl.kernel,
    out_type=jax.ShapeDtypeStruct((NC, N, F), jnp.float32),
    mesh=_mesh,
    scratch_types=[
        pltpu.VMEM((NB, BLK), jnp.int32),
        pltpu.VMEM((NB, BLK), jnp.int32),
        pltpu.VMEM((BLK, F), jnp.float32),
        pltpu.VMEM((BLK, F), jnp.float32),
        pltpu.VMEM_SHARED((N, F), jnp.float32),
        pltpu.SemaphoreType.DMA,
    ],
)(_edge_body)


RB = 1000
GRID = N // RB


def _k2_body(degp_ref, x_ref, w_ref, g_ref, dinv_ref):
    deg = degp_ref[0, :, 0:1] + degp_ref[1, :, 0:1] + 1.0
    dinv = lax.rsqrt(deg)
    h = jnp.dot(x_ref[...], w_ref[...], preferred_element_type=jnp.float32)
    g_ref[...] = h * dinv
    dinv_ref[...] = dinv


def _k2(degp, x, w):
    return pl.pallas_call(
        _k2_body,
        grid=(GRID,),
        in_specs=[
            pl.BlockSpec((NC, RB, DW), lambda i: (0, i, 0)),
            pl.BlockSpec((RB, F), lambda i: (i, 0)),
            pl.BlockSpec((F, F), lambda i: (0, 0)),
        ],
        out_specs=[
            pl.BlockSpec((RB, F), lambda i: (i, 0)),
            pl.BlockSpec((RB, 1), lambda i: (i, 0)),
        ],
        out_shape=[
            jax.ShapeDtypeStruct((N, F), jnp.float32),
            jax.ShapeDtypeStruct((N, 1), jnp.float32),
        ],
    )(degp, x, w)


def _k4_body(p_ref, g_ref, dinv_ref, b_ref, w_ref, g2_ref):
    sacc = p_ref[0] + p_ref[1] + g_ref[...]
    h = jnp.maximum(dinv_ref[...] * sacc + b_ref[...], 0.0)
    g2_ref[...] = jnp.dot(h, w_ref[...],
                          preferred_element_type=jnp.float32) * dinv_ref[...]


def _k4(p, g, dinv, b, w):
    return pl.pallas_call(
        _k4_body,
        grid=(GRID,),
        in_specs=[
            pl.BlockSpec((NC, RB, F), lambda i: (0, i, 0)),
            pl.BlockSpec((RB, F), lambda i: (i, 0)),
            pl.BlockSpec((RB, 1), lambda i: (i, 0)),
            pl.BlockSpec((1, F), lambda i: (0, 0)),
            pl.BlockSpec((F, F), lambda i: (0, 0)),
        ],
        out_specs=pl.BlockSpec((RB, F), lambda i: (i, 0)),
        out_shape=jax.ShapeDtypeStruct((N, F), jnp.float32),
    )(p, g, dinv, b, w)


def _k6_body(p_ref, g_ref, dinv_ref, b_ref, o_ref):
    sacc = p_ref[0] + p_ref[1] + g_ref[...]
    o_ref[...] = jnp.maximum(dinv_ref[...] * sacc + b_ref[...], 0.0)


def _k6(p, g, dinv, b):
    return pl.pallas_call(
        _k6_body,
        grid=(GRID,),
        in_specs=[
            pl.BlockSpec((NC, RB, F), lambda i: (0, i, 0)),
            pl.BlockSpec((RB, F), lambda i: (i, 0)),
            pl.BlockSpec((RB, 1), lambda i: (i, 0)),
            pl.BlockSpec((1, F), lambda i: (0, 0)),
        ],
        out_specs=pl.BlockSpec((RB, F), lambda i: (i, 0)),
        out_shape=jax.ShapeDtypeStruct((N, F), jnp.float32),
    )(p, g, dinv, b)


@jax.jit
def kernel(x, edge_index, W1, b1, W2, b2):
    src = edge_index[0].reshape(NW, NB, BLK)
    dst = edge_index[1].reshape(NW, NB, BLK)
    b1r = b1.reshape(1, F)
    b2r = b2.reshape(1, F)

    degp = _deg_call(dst)
    g1, dinv = _k2(degp, x, W1)
    p1 = _edge_call(src, dst, g1)
    g2 = _k4(p1, g1, dinv, b1r, W2)
    p2 = _edge_call(src, dst, g2)
    return _k6(p2, g2, dinv, b2r)


# speedup vs baseline: 20.1261x; 20.1261x over previous
import functools

import jax
import jax.numpy as jnp
from jax import lax
from jax.experimental import pallas as pl
from jax.experimental.pallas import tpu as pltpu
from jax.experimental.pallas import tpu_sc as plsc

N = 10000
F = 128
E = 320000
NC = 2
NS = 16
NW = NC * NS
EPT = E // NW
BLK = 80
NB = EPT // BLK
RPS = 624
TAIL = N - NS * RPS
DW = 16
ZR = 16

_mesh = plsc.VectorSubcoreMesh(core_axis_name="c", subcore_axis_name="s")


def _deg_body(dst_hbm, out_hbm, dst_v, ones_v, zeros_v, accd):
    c = lax.axis_index("c")
    s = lax.axis_index("s")
    wid = s * NC + c

    def fill(i, _):
        ones_v[i, :] = jnp.ones((16,), jnp.float32)
        zeros_v[i, :] = jnp.zeros((16,), jnp.float32)
        return 0

    lax.fori_loop(0, BLK, fill, 0)

    base = s * RPS
    for k in range(7):
        pltpu.sync_copy(zeros_v, accd.at[pl.ds(base + k * BLK, BLK)])
    pltpu.sync_copy(zeros_v.at[pl.ds(0, RPS - 7 * BLK)],
                    accd.at[pl.ds(base + 7 * BLK, RPS - 7 * BLK)])

    @pl.when(s == NS - 1)
    def _():
        pltpu.sync_copy(zeros_v.at[pl.ds(0, TAIL)],
                        accd.at[pl.ds(NS * RPS, TAIL)])

    plsc.subcore_barrier()

    pltpu.sync_copy(dst_hbm.at[wid], dst_v)

    def body(j, _):
        pltpu.sync_copy(ones_v, accd.at[dst_v.at[j]], add=True)
        return 0

    lax.fori_loop(0, NB, body, 0)
    plsc.subcore_barrier()
    pltpu.sync_copy(accd.at[pl.ds(base, RPS)], out_hbm.at[c, pl.ds(base, RPS)])

    @pl.when(s == NS - 1)
    def _():
        pltpu.sync_copy(accd.at[pl.ds(NS * RPS, TAIL)],
                        out_hbm.at[c, pl.ds(NS * RPS, TAIL)])


_deg_call = functools.partial(
    pl.kernel,
    out_type=jax.ShapeDtypeStruct((NC, N, DW), jnp.float32),
    mesh=_mesh,
    scratch_types=[
        pltpu.VMEM((NB, BLK), jnp.int32),
        pltpu.VMEM((BLK, DW), jnp.float32),
        pltpu.VMEM((BLK, DW), jnp.float32),
        pltpu.VMEM_SHARED((N, DW), jnp.float32),
    ],
)(_deg_body)


def _edge_body(src_hbm, dst_hbm, g_hbm, out_hbm, src_v, dst_v, rows, zrows, acc,
               sem):
    c = lax.axis_index("c")
    s = lax.axis_index("s")
    wid = s * NC + c

    def zfill(r, _):
        for k in range(F // 16):
            zrows[r, pl.ds(k * 16, 16)] = jnp.zeros((16,), jnp.float32)
        return 0

    lax.fori_loop(0, ZR, zfill, 0)

    base = s * RPS
    for k in range(RPS // ZR):
        pltpu.sync_copy(zrows, acc.at[pl.ds(base + k * ZR, ZR)])

    @pl.when(s == NS - 1)
    def _():
        pltpu.sync_copy(zrows.at[pl.ds(0, TAIL)],
                        acc.at[pl.ds(NS * RPS, TAIL)])

    pltpu.sync_copy(src_hbm.at[wid], src_v)
    pltpu.sync_copy(dst_hbm.at[wid], dst_v)
    plsc.subcore_barrier()

    def body(j, _):
        pltpu.async_copy(g_hbm.at[src_v.at[j]], rows, sem).wait()
        pltpu.sync_copy(rows, acc.at[dst_v.at[j]], add=True)
        return 0

    lax.fori_loop(0, NB, body, 0)
    plsc.subcore_barrier()
    pltpu.sync_copy(acc.at[pl.ds(base, RPS)], out_hbm.at[c, pl.ds(base, RPS)])

    @pl.when(s == NS - 1)
    def _():
        pltpu.sync_copy(acc.at[pl.ds(NS * RPS, TAIL)],
                        out_hbm.at[c, pl.ds(NS * RPS, TAIL)])


_edge_call = functools.partial(
    pl.kernel,
    out_type=jax.ShapeDtypeStruct((NC, N, F), jnp.float32),
    mesh=_mesh,
    scratch_types=[
        pltpu.VMEM((NB, BLK), jnp.int32),
        pltpu.VMEM((NB, BLK), jnp.int32),
        pltpu.VMEM((BLK, F), jnp.float32),
        pltpu.VMEM((ZR, F), jnp.float32),
        pltpu.VMEM_SHARED((N, F), jnp.float32),
        pltpu.SemaphoreType.DMA,
    ],
)(_edge_body)


RB = 1000
GRID = N // RB


def _k2_body(degp_ref, x_ref, w_ref, g_ref, dinv_ref):
    deg = degp_ref[0, :, 0:1] + degp_ref[1, :, 0:1] + 1.0
    dinv = lax.rsqrt(deg)
    h = jnp.dot(x_ref[...], w_ref[...], preferred_element_type=jnp.float32)
    g_ref[...] = h * dinv
    dinv_ref[...] = dinv


def _k2(degp, x, w):
    return pl.pallas_call(
        _k2_body,
        grid=(GRID,),
        in_specs=[
            pl.BlockSpec((NC, RB, DW), lambda i: (0, i, 0)),
            pl.BlockSpec((RB, F), lambda i: (i, 0)),
            pl.BlockSpec((F, F), lambda i: (0, 0)),
        ],
        out_specs=[
            pl.BlockSpec((RB, F), lambda i: (i, 0)),
            pl.BlockSpec((RB, 1), lambda i: (i, 0)),
        ],
        out_shape=[
            jax.ShapeDtypeStruct((N, F), jnp.float32),
            jax.ShapeDtypeStruct((N, 1), jnp.float32),
        ],
    )(degp, x, w)


def _k4_body(p_ref, g_ref, dinv_ref, b_ref, w_ref, g2_ref):
    sacc = p_ref[0] + p_ref[1] + g_ref[...]
    h = jnp.maximum(dinv_ref[...] * sacc + b_ref[...], 0.0)
    g2_ref[...] = jnp.dot(h, w_ref[...],
                          preferred_element_type=jnp.float32) * dinv_ref[...]


def _k4(p, g, dinv, b, w):
    return pl.pallas_call(
        _k4_body,
        grid=(GRID,),
        in_specs=[
            pl.BlockSpec((NC, RB, F), lambda i: (0, i, 0)),
            pl.BlockSpec((RB, F), lambda i: (i, 0)),
            pl.BlockSpec((RB, 1), lambda i: (i, 0)),
            pl.BlockSpec((1, F), lambda i: (0, 0)),
            pl.BlockSpec((F, F), lambda i: (0, 0)),
        ],
        out_specs=pl.BlockSpec((RB, F), lambda i: (i, 0)),
        out_shape=jax.ShapeDtypeStruct((N, F), jnp.float32),
    )(p, g, dinv, b, w)


def _k6_body(p_ref, g_ref, dinv_ref, b_ref, o_ref):
    sacc = p_ref[0] + p_ref[1] + g_ref[...]
    o_ref[...] = jnp.maximum(dinv_ref[...] * sacc + b_ref[...], 0.0)


def _k6(p, g, dinv, b):
    return pl.pallas_call(
        _k6_body,
        grid=(GRID,),
        in_specs=[
            pl.BlockSpec((NC, RB, F), lambda i: (0, i, 0)),
            pl.BlockSpec((RB, F), lambda i: (i, 0)),
            pl.BlockSpec((RB, 1), lambda i: (i, 0)),
            pl.BlockSpec((1, F), lambda i: (0, 0)),
        ],
        out_specs=pl.BlockSpec((RB, F), lambda i: (i, 0)),
        out_shape=jax.ShapeDtypeStruct((N, F), jnp.float32),
    )(p, g, dinv, b)


@jax.jit
def kernel(x, edge_index, W1, b1, W2, b2):
    src = edge_index[0].reshape(NW, NB, BLK)
    dst = edge_index[1].reshape(NW, NB, BLK)
    b1r = b1.reshape(1, F)
    b2r = b2.reshape(1, F)

    degp = _deg_call(dst)
    g1, dinv = _k2(degp, x, W1)
    p1 = _edge_call(src, dst, g1)
    g2 = _k4(p1, g1, dinv, b1r, W2)
    p2 = _edge_call(src, dst, g2)
    return _k6(p2, g2, dinv, b2r)


# speedup vs baseline: 25.2360x; 1.2539x over previous
import functools

import jax
import jax.numpy as jnp
from jax import lax
from jax.experimental import pallas as pl
from jax.experimental.pallas import tpu as pltpu
from jax.experimental.pallas import tpu_sc as plsc

N = 10000
F = 128
E = 320000
NC = 2
NS = 16
NW = NC * NS
EPT = E // NW
BLK = 80
NB = EPT // BLK
CH = 25
NCHUNK = NB // CH
RPS = 624
TAIL = N - NS * RPS
DW = 16

_mesh = plsc.VectorSubcoreMesh(core_axis_name="c", subcore_axis_name="s")


def _deg_body(dst_hbm, out_hbm, dst_v, ones_v, zeros_v, accd):
    c = lax.axis_index("c")
    s = lax.axis_index("s")
    wid = s * NC + c

    def fill(i, _):
        ones_v[i, :] = jnp.ones((16,), jnp.float32)
        zeros_v[i, :] = jnp.zeros((16,), jnp.float32)
        return 0

    lax.fori_loop(0, BLK, fill, 0)

    base = s * RPS
    for k in range(RPS // BLK):
        pltpu.sync_copy(zeros_v, accd.at[pl.ds(base + k * BLK, BLK)])
    pltpu.sync_copy(zeros_v.at[pl.ds(0, RPS % BLK)],
                    accd.at[pl.ds(base + (RPS // BLK) * BLK, RPS % BLK)])

    @pl.when(s == NS - 1)
    def _():
        pltpu.sync_copy(zeros_v.at[pl.ds(0, TAIL)],
                        accd.at[pl.ds(NS * RPS, TAIL)])

    plsc.subcore_barrier()

    pltpu.sync_copy(dst_hbm.at[wid], dst_v)

    def body(j, _):
        pltpu.sync_copy(ones_v, accd.at[dst_v.at[j]], add=True)
        return 0

    lax.fori_loop(0, NB, body, 0)
    plsc.subcore_barrier()
    pltpu.sync_copy(accd.at[pl.ds(base, RPS)], out_hbm.at[c, pl.ds(base, RPS)])

    @pl.when(s == NS - 1)
    def _():
        pltpu.sync_copy(accd.at[pl.ds(NS * RPS, TAIL)],
                        out_hbm.at[c, pl.ds(NS * RPS, TAIL)])


_deg_call = functools.partial(
    pl.kernel,
    out_type=jax.ShapeDtypeStruct((NC, N, DW), jnp.float32),
    mesh=_mesh,
    scratch_types=[
        pltpu.VMEM((NB, BLK), jnp.int32),
        pltpu.VMEM((BLK, DW), jnp.float32),
        pltpu.VMEM((BLK, DW), jnp.float32),
        pltpu.VMEM_SHARED((N, DW), jnp.float32),
    ],
)(_deg_body)


def _edge_body(src_hbm, dst_hbm, g_hbm, out_hbm, src_v, dstc, rows, acc,
               sem_g, sem_i):
    c = lax.axis_index("c")
    s = lax.axis_index("s")
    wid = s * NC + c

    def zfill(r, _):
        for k in range(F // 16):
            rows[0, r, pl.ds(k * 16, 16)] = jnp.zeros((16,), jnp.float32)
        return 0

    lax.fori_loop(0, BLK, zfill, 0)

    base = s * RPS
    for k in range(RPS // BLK):
        pltpu.sync_copy(rows.at[0], acc.at[pl.ds(base + k * BLK, BLK)])
    pltpu.sync_copy(rows.at[0].at[pl.ds(0, RPS % BLK)],
                    acc.at[pl.ds(base + (RPS // BLK) * BLK, RPS % BLK)])

    @pl.when(s == NS - 1)
    def _():
        pltpu.sync_copy(rows.at[0].at[pl.ds(0, TAIL)],
                        acc.at[pl.ds(NS * RPS, TAIL)])

    pltpu.sync_copy(src_hbm.at[wid], src_v)
    pltpu.sync_copy(dst_hbm.at[wid, 0], dstc.at[0])
    plsc.subcore_barrier()

    pltpu.async_copy(g_hbm.at[src_v.at[0]], rows.at[0], sem_g.at[0])

    def body(j, _):
        b = lax.rem(j, 2)
        nb = lax.rem(j + 1, 2)
        cdiv = lax.div(j, CH)
        crow = lax.rem(j, CH)
        cbuf = lax.rem(cdiv, 2)

        pltpu.make_async_copy(g_hbm.at[src_v.at[j]], rows.at[b],
                              sem_g.at[b]).wait()

        @pl.when(jnp.logical_and(crow == 0, cdiv >= 1))
        def _():
            pltpu.make_async_copy(dst_hbm.at[wid, cdiv], dstc.at[cbuf],
                                  sem_i).wait()

        @pl.when(j + 1 < NB)
        def _():
            pltpu.async_copy(g_hbm.at[src_v.at[j + 1]], rows.at[nb],
                             sem_g.at[nb])

        @pl.when(jnp.logical_and(crow == 0, cdiv + 1 < NCHUNK))
        def _():
            pltpu.async_copy(dst_hbm.at[wid, cdiv + 1], dstc.at[1 - cbuf],
                             sem_i)

        pltpu.sync_copy(rows.at[b], acc.at[dstc.at[cbuf].at[crow]], add=True)

        return 0

    lax.fori_loop(0, NB, body, 0)
    plsc.subcore_barrier()
    pltpu.sync_copy(acc.at[pl.ds(base, RPS)], out_hbm.at[c, pl.ds(base, RPS)])

    @pl.when(s == NS - 1)
    def _():
        pltpu.sync_copy(acc.at[pl.ds(NS * RPS, TAIL)],
                        out_hbm.at[c, pl.ds(NS * RPS, TAIL)])


_edge_call = functools.partial(
    pl.kernel,
    out_type=jax.ShapeDtypeStruct((NC, N, F), jnp.float32),
    mesh=_mesh,
    scratch_types=[
        pltpu.VMEM((NB, BLK), jnp.int32),
        pltpu.VMEM((2, CH, BLK), jnp.int32),
        pltpu.VMEM((2, BLK, F), jnp.float32),
        pltpu.VMEM_SHARED((N, F), jnp.float32),
        pltpu.SemaphoreType.DMA((2,)),
        pltpu.SemaphoreType.DMA,
    ],
)(_edge_body)


RB = 1000
GRID = N // RB


def _k2_body(degp_ref, x_ref, w_ref, g_ref, dinv_ref):
    deg = degp_ref[0, :, 0:1] + degp_ref[1, :, 0:1] + 1.0
    dinv = lax.rsqrt(deg)
    h = jnp.dot(x_ref[...], w_ref[...], preferred_element_type=jnp.float32)
    g_ref[...] = h * dinv
    dinv_ref[...] = dinv


def _k2(degp, x, w):
    return pl.pallas_call(
        _k2_body,
        grid=(GRID,),
        in_specs=[
            pl.BlockSpec((NC, RB, DW), lambda i: (0, i, 0)),
            pl.BlockSpec((RB, F), lambda i: (i, 0)),
            pl.BlockSpec((F, F), lambda i: (0, 0)),
        ],
        out_specs=[
            pl.BlockSpec((RB, F), lambda i: (i, 0)),
            pl.BlockSpec((RB, 1), lambda i: (i, 0)),
        ],
        out_shape=[
            jax.ShapeDtypeStruct((N, F), jnp.float32),
            jax.ShapeDtypeStruct((N, 1), jnp.float32),
        ],
    )(degp, x, w)


def _k4_body(p_ref, g_ref, dinv_ref, b_ref, w_ref, g2_ref):
    sacc = p_ref[0] + p_ref[1] + g_ref[...]
    h = jnp.maximum(dinv_ref[...] * sacc + b_ref[...], 0.0)
    g2_ref[...] = jnp.dot(h, w_ref[...],
                          preferred_element_type=jnp.float32) * dinv_ref[...]


def _k4(p, g, dinv, b, w):
    return pl.pallas_call(
        _k4_body,
        grid=(GRID,),
        in_specs=[
            pl.BlockSpec((NC, RB, F), lambda i: (0, i, 0)),
            pl.BlockSpec((RB, F), lambda i: (i, 0)),
            pl.BlockSpec((RB, 1), lambda i: (i, 0)),
            pl.BlockSpec((1, F), lambda i: (0, 0)),
            pl.BlockSpec((F, F), lambda i: (0, 0)),
        ],
        out_specs=pl.BlockSpec((RB, F), lambda i: (i, 0)),
        out_shape=jax.ShapeDtypeStruct((N, F), jnp.float32),
    )(p, g, dinv, b, w)


def _k6_body(p_ref, g_ref, dinv_ref, b_ref, o_ref):
    sacc = p_ref[0] + p_ref[1] + g_ref[...]
    o_ref[...] = jnp.maximum(dinv_ref[...] * sacc + b_ref[...], 0.0)


def _k6(p, g, dinv, b):
    return pl.pallas_call(
        _k6_body,
        grid=(GRID,),
        in_specs=[
            pl.BlockSpec((NC, RB, F), lambda i: (0, i, 0)),
            pl.BlockSpec((RB, F), lambda i: (i, 0)),
            pl.BlockSpec((RB, 1), lambda i: (i, 0)),
            pl.BlockSpec((1, F), lambda i: (0, 0)),
        ],
        out_specs=pl.BlockSpec((RB, F), lambda i: (i, 0)),
        out_shape=jax.ShapeDtypeStruct((N, F), jnp.float32),
    )(p, g, dinv, b)


@jax.jit
def kernel(x, edge_index, W1, b1, W2, b2):
    src = edge_index[0].reshape(NW, NB, BLK)
    dst = edge_index[1].reshape(NW, NB, BLK)
    dst4 = edge_index[1].reshape(NW, NCHUNK, CH, BLK)
    b1r = b1.reshape(1, F)
    b2r = b2.reshape(1, F)

    degp = _deg_call(dst)
    g1, dinv = _k2(degp, x, W1)
    p1 = _edge_call(src, dst4, g1)
    g2 = _k4(p1, g1, dinv, b1r, W2)
    p2 = _edge_call(src, dst4, g2)
    return _k6(p2, g2, dinv, b2r)
